# baseline (speedup 1.0000x reference)
import functools

import jax
import jax.numpy as jnp
from jax import lax
from jax.experimental import pallas as pl
from jax.experimental.pallas import tpu as pltpu
from jax.experimental.pallas import tpu_sc as plsc

F32 = jnp.float32
HI = lax.Precision.HIGHEST

N = 10000
NP = 10240
E = 160000
DIN = 256
DH = 128
HEADS = 8
DOUT = 64

CH = 80
NCK = E // (16 * CH)
ROWS_T = NP // 16
BN = 1280
G = NP // BN
NV = 16
FG = 16
FW = 64


def _colstats(a):
    H, n, D = a.shape

    def body(a_ref, o_ref):
        i = pl.program_id(0)

        @pl.when(i == 0)
        def _():
            o_ref[...] = jnp.zeros_like(o_ref)

        ab = a_ref[...]
        o_ref[0] += jnp.sum(ab, axis=1)
        o_ref[1] += jnp.sum(ab * ab, axis=1)

    return pl.pallas_call(
        body,
        grid=(n // BN,),
        in_specs=[pl.BlockSpec((H, BN, D), lambda i: (0, i, 0))],
        out_specs=pl.BlockSpec((2, H, D), lambda i: (0, 0, 0)),
        out_shape=jax.ShapeDtypeStruct((2, H, D), F32),
    )(a)


def _mm1(x, scale1, shift1, W1, a1s, a1d):
    def body(x_ref, sc_ref, sh_ref, w_ref, as_ref, ad_ref, hp_ref, e_ref):
        i = pl.program_id(0)
        hb = x_ref[...] * sc_ref[...] + sh_ref[...]
        for h in range(HEADS):
            wh = w_ref[:, h * DH:(h + 1) * DH]
            hph = lax.dot_general(hb, wh, (((1,), (0,)), ((), ())), precision=HI)
            hp_ref[2 * h] = hph[:, :FW]
            hp_ref[2 * h + 1] = hph[:, FW:]
            e_ref[h:h + 1, pl.ds(i * BN, BN)] = lax.dot_general(
                as_ref[h:h + 1, :], hph, (((1,), (1,)), ((), ())), precision=HI)
            e_ref[h + HEADS:h + HEADS + 1, pl.ds(i * BN, BN)] = lax.dot_general(
                ad_ref[h:h + 1, :], hph, (((1,), (1,)), ((), ())), precision=HI)

    return pl.pallas_call(
        body,
        grid=(G,),
        in_specs=[
            pl.BlockSpec((BN, DIN), lambda i: (i, 0)),
            pl.BlockSpec((1, DIN), lambda i: (0, 0)),
            pl.BlockSpec((1, DIN), lambda i: (0, 0)),
            pl.BlockSpec((DIN, HEADS * DH), lambda i: (0, 0)),
            pl.BlockSpec((HEADS, DH), lambda i: (0, 0)),
            pl.BlockSpec((HEADS, DH), lambda i: (0, 0)),
        ],
        out_specs=[
            pl.BlockSpec((FG, BN, FW), lambda i: (0, i, 0)),
            pl.BlockSpec((2 * HEADS, NP), lambda i: (0, 0)),
        ],
        out_shape=[
            jax.ShapeDtypeStruct((FG, NP, FW), F32),
            jax.ShapeDtypeStruct((2 * HEADS, NP), F32),
        ],
    )(x, scale1, shift1, W1, a1s, a1d)


def _mm2(h2, scale2, shift2, W2, a2):
    def body(h_ref, sc_ref, sh_ref, w_ref, a_ref, hp_ref, e_ref):
        i = pl.program_id(0)
        acc = jnp.zeros((BN, DOUT), F32)
        for f in range(FG):
            yb = h_ref[f] * sc_ref[f:f + 1, :] + sh_ref[f:f + 1, :]
            yb = jnp.maximum(yb, 0.01 * yb)
            acc = acc + lax.dot_general(
                yb, w_ref[f * FW:(f + 1) * FW, :], (((1,), (0,)), ((), ())),
                precision=HI)
        hp_ref[...] = acc
        e_ref[:, pl.ds(i * BN, BN)] = lax.dot_general(
            a_ref[...], acc, (((1,), (1,)), ((), ())), precision=HI)

    return pl.pallas_call(
        body,
        grid=(G,),
        in_specs=[
            pl.BlockSpec((FG, BN, FW), lambda i: (0, i, 0)),
            pl.BlockSpec((FG, FW), lambda i: (0, 0)),
            pl.BlockSpec((FG, FW), lambda i: (0, 0)),
            pl.BlockSpec((HEADS * DH, DOUT), lambda i: (0, 0)),
            pl.BlockSpec((2, DOUT), lambda i: (0, 0)),
        ],
        out_specs=[
            pl.BlockSpec((BN, DOUT), lambda i: (i, 0)),
            pl.BlockSpec((2, NP), lambda i: (0, 0)),
        ],
        out_shape=[
            jax.ShapeDtypeStruct((NP, DOUT), F32),
            jax.ShapeDtypeStruct((2, NP), F32),
        ],
    )(h2, scale2, shift2, W2, a2)


def _final(accA, accB, W_fc, b_fc):
    def body(a_ref, b_ref, w_ref, bias_ref, o_ref, h_ref):
        hf = a_ref[...] + b_ref[...]
        h_ref[...] = hf
        o_ref[...] = lax.dot_general(
            jnp.maximum(hf, 0.0), w_ref[...], (((1,), (0,)), ((), ())),
            precision=HI) + bias_ref[...]

    return pl.pallas_call(
        body,
        grid=(G,),
        in_specs=[
            pl.BlockSpec((BN, DOUT), lambda i: (i, 0)),
            pl.BlockSpec((BN, DOUT), lambda i: (i, 0)),
            pl.BlockSpec((DOUT, 2), lambda i: (0, 0)),
            pl.BlockSpec((1, 2), lambda i: (0, 0)),
        ],
        out_specs=[
            pl.BlockSpec((BN, 2), lambda i: (i, 0)),
            pl.BlockSpec((BN, DOUT), lambda i: (i, 0)),
        ],
        out_shape=[
            jax.ShapeDtypeStruct((NP, 2), F32),
            jax.ShapeDtypeStruct((NP, DOUT), F32),
        ],
    )(accA, accB, W_fc, b_fc)


def _leaky_exp(t):
    return jnp.exp(jnp.maximum(t, 0.2 * t))


def _init_ident(ident):
    def body(k, _):
        def vv(i, _):
            ident[k, pl.ds(i * NV, NV)] = (
                jnp.arange(NV, dtype=jnp.int32) + i * NV + k * 128)
            return 0
        lax.fori_loop(0, 8, vv, 0)
        return 0
    lax.fori_loop(0, 5, body, 0)


def _den_phase(s, src3, dst3, es_tab, ed_tab, den_buf, den_sh, ident):
    zv = jnp.zeros((NV,), F32)

    def zinit(i, _):
        den_buf[i, :] = zv
        return 0
    lax.fori_loop(0, NP // 16, zinit, 0)

    def chunk(j, _):
        for k in range(CH // NV):
            sv = src3[j, pl.ds(k * NV, NV)]
            dv = dst3[j, pl.ds(k * NV, NV)]
            es = plsc.load_gather(es_tab, [sv])
            ed = plsc.load_gather(ed_tab, [dv])
            z = _leaky_exp(es + ed)
            plsc.addupdate_scatter(
                den_buf, [lax.shift_right_logical(dv, 4),
                          lax.bitwise_and(dv, 15)], z)
        return 0
    lax.fori_loop(0, NCK, chunk, 0)

    @pl.when(s == 0)
    def _():
        pltpu.sync_copy(den_buf, den_sh)
    plsc.subcore_barrier()

    @pl.when(s != 0)
    def _():
        for k in range(5):
            pltpu.sync_copy(den_buf.at[pl.ds(k * 128, 128)],
                            den_sh.at[ident.at[k]], add=True)
    plsc.subcore_barrier()
    pltpu.sync_copy(den_sh, den_buf)


def _alpha_phase(src3, dst3, es_tab, ed_tab, den_buf, alpha_all):
    @plsc.parallel_loop(0, NCK, 1, unroll=2)
    def _(j):
        for k in range(CH // NV):
            sv = src3[j, pl.ds(k * NV, NV)]
            dv = dst3[j, pl.ds(k * NV, NV)]
            es = plsc.load_gather(es_tab, [sv])
            ed = plsc.load_gather(ed_tab, [dv])
            z = _leaky_exp(es + ed)
            den = plsc.load_gather(
                den_buf, [lax.shift_right_logical(dv, 4),
                          lax.bitwise_and(dv, 15)])
            alpha_all[j, pl.ds(k * NV, NV)] = z / (den + 1e-16)


def _zero_rows(rows):
    zv = jnp.zeros((NV,), F32)

    @plsc.parallel_loop(0, CH, 1, unroll=8)
    def _(i):
        for r in range(FW // NV):
            rows[i, pl.ds(r * NV, NV)] = zv


def _zero_acc_stripe(s, rows, acc_sh):
    base = s * ROWS_T
    for k in range(ROWS_T // CH):
        pltpu.sync_copy(rows, acc_sh.at[pl.ds(base + k * CH, CH)])


def _agg_chunk(j, hp_f, src3, dst3, alpha_all, rows, acc_sh, sem):
    pltpu.async_copy(hp_f.at[src3.at[j]], rows, sem).wait()
    _mul_scatter(j, rows, src3, dst3, alpha_all, acc_sh)


def _mul_scatter(j, rows, src3, dst3, alpha_all, acc_sh):
    _mul_rows(j, rows, alpha_all)
    pltpu.sync_copy(rows, acc_sh.at[dst3.at[j]], add=True)


def _mul_rows(j, rows, alpha_all):
    jv = jnp.full((NV,), j, jnp.int32)

    @plsc.parallel_loop(0, CH, 1, unroll=4)
    def _(i):
        a = plsc.load_gather(alpha_all, [jv, jnp.full((NV,), i, jnp.int32)])
        for r in range(FW // NV):
            rows[i, pl.ds(r * NV, NV)] = rows[i, pl.ds(r * NV, NV)] * a


def _agg_pipe(hp_f, src3, dst3, alpha_all, rr, acc_sh, gs, ss):
    pltpu.async_copy(hp_f.at[src3.at[0]], rr[0], gs[0])

    def substep(j, b, tail):
        bn = (b + 1) % 3
        pltpu.make_async_copy(hp_f.at[src3.at[j]], rr[b], gs[b]).wait()

        @pl.when(j >= 2)
        def _():
            jm2 = jnp.maximum(j - 2, 0)
            pltpu.make_async_copy(rr[bn], acc_sh.at[dst3.at[jm2]],
                                  ss[bn]).wait()
        if not tail:
            pltpu.async_copy(hp_f.at[src3.at[j + 1]], rr[bn], gs[bn])
        _mul_rows(j, rr[b], alpha_all)
        pltpu.async_copy(rr[b], acc_sh.at[dst3.at[j]], ss[b], add=True)

    def step(t, _):
        j = 3 * t
        substep(j, 0, False)
        substep(j + 1, 1, False)
        substep(j + 2, 2, False)
        return 0
    lax.fori_loop(0, (NCK - 2) // 3, step, 0)
    substep(NCK - 2, (NCK - 2) % 3, False)
    substep(NCK - 1, (NCK - 1) % 3, True)
    for j in (NCK - 2, NCK - 1):
        b = j % 3
        pltpu.make_async_copy(rr[b], acc_sh.at[dst3.at[j]], ss[b]).wait()


def _sc_scratch():
    return [
        pltpu.VMEM((NCK, CH), jnp.int32),
        pltpu.VMEM((NCK, CH), jnp.int32),
        pltpu.VMEM((NP,), F32),
        pltpu.VMEM((NP,), F32),
        pltpu.VMEM((NP // 16, 16), F32),
        pltpu.VMEM((5, 128), jnp.int32),
        pltpu.VMEM((NCK, CH), F32),
        pltpu.VMEM((CH, FW), F32),
        pltpu.VMEM((CH, FW), F32),
        pltpu.VMEM((CH, FW), F32),
        pltpu.VMEM_SHARED((NP // 16, 16), F32),
        pltpu.VMEM_SHARED((NP, FW), F32),
        pltpu.SemaphoreType.DMA,
        pltpu.SemaphoreType.DMA,
        pltpu.SemaphoreType.DMA,
        pltpu.SemaphoreType.DMA,
        pltpu.SemaphoreType.DMA,
        pltpu.SemaphoreType.DMA,
    ]


_SC_PARAMS = pltpu.CompilerParams(use_tc_tiling_on_sc=False,
                                  needs_layout_passes=False)


def _make_sc1():
    mesh = plsc.VectorSubcoreMesh(core_axis_name="c", subcore_axis_name="s")

    @functools.partial(
        pl.kernel,
        out_type=jax.ShapeDtypeStruct((FG, NP, FW), F32),
        mesh=mesh,
        scratch_types=_sc_scratch(),
        compiler_params=_SC_PARAMS,
    )
    def sc1(src2d, dst2d, e1T, hp, out, src3, dst3, es_tab, ed_tab, den_buf,
            ident, alpha_all, rows0, rows1, rows2, den_sh, acc_sh, g0, g1, g2,
            s0, s1, s2):
        c = lax.axis_index("c")
        s = lax.axis_index("s")

        pltpu.sync_copy(src2d.at[pl.ds(s * NCK, NCK)], src3)
        pltpu.sync_copy(dst2d.at[pl.ds(s * NCK, NCK)], dst3)
        _init_ident(ident)

        def head(hh, _):
            h = c * 4 + hh
            pltpu.sync_copy(e1T.at[h], es_tab)
            pltpu.sync_copy(e1T.at[h + HEADS], ed_tab)
            _den_phase(s, src3, dst3, es_tab, ed_tab, den_buf, den_sh, ident)
            _alpha_phase(src3, dst3, es_tab, ed_tab, den_buf, alpha_all)
            for half in range(2):
                f = 2 * h + half
                _zero_rows(rows0)
                _zero_acc_stripe(s, rows0, acc_sh)
                plsc.subcore_barrier()
                _agg_pipe(hp.at[f], src3, dst3, alpha_all,
                          (rows0, rows1, rows2), acc_sh, (g0, g1, g2),
                          (s0, s1, s2))
                plsc.subcore_barrier()
                pltpu.sync_copy(
                    acc_sh.at[pl.ds(s * ROWS_T, ROWS_T)],
                    out.at[f].at[pl.ds(s * ROWS_T, ROWS_T)])
                plsc.subcore_barrier()
            return 0
        lax.fori_loop(0, 4, head, 0)

    return sc1


def _make_sc2():
    mesh = plsc.VectorSubcoreMesh(core_axis_name="c", subcore_axis_name="s")

    @functools.partial(
        pl.kernel,
        out_type=(jax.ShapeDtypeStruct((NP, DOUT), F32),
                  jax.ShapeDtypeStruct((NP, DOUT), F32)),
        mesh=mesh,
        scratch_types=_sc_scratch(),
        compiler_params=_SC_PARAMS,
    )
    def sc2(src2d, dst2d, e2T, hp2, outA, outB, src3, dst3, es_tab, ed_tab,
            den_buf, ident, alpha_all, rows0, rows1, rows2, den_sh, acc_sh,
            g0, g1, g2, s0, s1, s2):
        c = lax.axis_index("c")
        s = lax.axis_index("s")

        pltpu.sync_copy(src2d.at[pl.ds(s * NCK, NCK)], src3)
        pltpu.sync_copy(dst2d.at[pl.ds(s * NCK, NCK)], dst3)
        _init_ident(ident)

        pltpu.sync_copy(e2T.at[0], es_tab)
        pltpu.sync_copy(e2T.at[1], ed_tab)
        _den_phase(s, src3, dst3, es_tab, ed_tab, den_buf, den_sh, ident)
        _alpha_phase(src3, dst3, es_tab, ed_tab, den_buf, alpha_all)
        _zero_rows(rows0)
        _zero_acc_stripe(s, rows0, acc_sh)
        plsc.subcore_barrier()

        def chunk(j, _):
            @pl.when(lax.rem(j, 2) == c)
            def _():
                _agg_chunk(j, hp2, src3, dst3, alpha_all, rows0, acc_sh, g0)
            return 0
        lax.fori_loop(0, NCK, chunk, 0)
        plsc.subcore_barrier()

        @pl.when(c == 0)
        def _():
            pltpu.sync_copy(acc_sh.at[pl.ds(s * ROWS_T, ROWS_T)],
                            outA.at[pl.ds(s * ROWS_T, ROWS_T)])

        @pl.when(c == 1)
        def _():
            pltpu.sync_copy(acc_sh.at[pl.ds(s * ROWS_T, ROWS_T)],
                            outB.at[pl.ds(s * ROWS_T, ROWS_T)])

    return sc2


def kernel(x, adj, gamma1, beta1, W1, a1_src, a1_dst, gamma2, beta2, W2,
           a2_src, a2_dst, W_fc, b_fc):
    src2d = adj[0].reshape(E // CH, CH)
    dst2d = adj[1].reshape(E // CH, CH)
    xp = jnp.pad(x, ((0, NP - N), (0, 0)))

    st1 = _colstats(xp.reshape(1, NP, DIN))
    mean1 = st1[0, 0] / N
    var1 = st1[1, 0] / N - mean1 * mean1
    scale1 = gamma1 * lax.rsqrt(var1 + 1e-5)
    shift1 = beta1 - mean1 * scale1

    hp1, e1T = _mm1(xp, scale1.reshape(1, DIN), shift1.reshape(1, DIN), W1,
                    a1_src, a1_dst)
    h2 = _make_sc1()(src2d, dst2d, e1T, hp1)

    st2 = _colstats(h2)
    mean2 = st2[0] / N
    var2 = st2[1] / N - mean2 * mean2
    scale2 = gamma2.reshape(FG, FW) * lax.rsqrt(var2 + 1e-5)
    shift2 = beta2.reshape(FG, FW) - mean2 * scale2

    hp2, e2T = _mm2(h2, scale2, shift2, W2,
                    jnp.concatenate([a2_src, a2_dst], axis=0))
    accA, accB = _make_sc2()(src2d, dst2d, e2T, hp2)

    out, hfin = _final(accA, accB, W_fc, b_fc.reshape(1, 2))
    return (out[:N], hfin[:N])


# speedup vs baseline: 1.0004x; 1.0004x over previous
import functools

import jax
import jax.numpy as jnp
from jax import lax
from jax.experimental import pallas as pl
from jax.experimental.pallas import tpu as pltpu
from jax.experimental.pallas import tpu_sc as plsc

F32 = jnp.float32
HI = lax.Precision.HIGHEST

N = 10000
NP = 10240
E = 160000
DIN = 256
DH = 128
HEADS = 8
DOUT = 64

CH = 80
NCK = E // (16 * CH)
ROWS_T = NP // 16
BN = 1280
G = NP // BN
NV = 16
FG = 16
FW = 64


def _colstats(a):
    H, n, D = a.shape

    def body(a_ref, o_ref):
        i = pl.program_id(0)

        @pl.when(i == 0)
        def _():
            o_ref[...] = jnp.zeros_like(o_ref)

        ab = a_ref[...]
        o_ref[0] += jnp.sum(ab, axis=1)
        o_ref[1] += jnp.sum(ab * ab, axis=1)

    return pl.pallas_call(
        body,
        grid=(n // BN,),
        in_specs=[pl.BlockSpec((H, BN, D), lambda i: (0, i, 0))],
        out_specs=pl.BlockSpec((2, H, D), lambda i: (0, 0, 0)),
        out_shape=jax.ShapeDtypeStruct((2, H, D), F32),
    )(a)


def _mm1(x, scale1, shift1, W1, a1s, a1d):
    def body(x_ref, sc_ref, sh_ref, w_ref, as_ref, ad_ref, hp_ref, e_ref):
        i = pl.program_id(0)
        hb = x_ref[...] * sc_ref[...] + sh_ref[...]
        for h in range(HEADS):
            wh = w_ref[:, h * DH:(h + 1) * DH]
            hph = lax.dot_general(hb, wh, (((1,), (0,)), ((), ())), precision=HI)
            hp_ref[2 * h] = hph[:, :FW]
            hp_ref[2 * h + 1] = hph[:, FW:]
            e_ref[h:h + 1, pl.ds(i * BN, BN)] = lax.dot_general(
                as_ref[h:h + 1, :], hph, (((1,), (1,)), ((), ())), precision=HI)
            e_ref[h + HEADS:h + HEADS + 1, pl.ds(i * BN, BN)] = lax.dot_general(
                ad_ref[h:h + 1, :], hph, (((1,), (1,)), ((), ())), precision=HI)

    return pl.pallas_call(
        body,
        grid=(G,),
        in_specs=[
            pl.BlockSpec((BN, DIN), lambda i: (i, 0)),
            pl.BlockSpec((1, DIN), lambda i: (0, 0)),
            pl.BlockSpec((1, DIN), lambda i: (0, 0)),
            pl.BlockSpec((DIN, HEADS * DH), lambda i: (0, 0)),
            pl.BlockSpec((HEADS, DH), lambda i: (0, 0)),
            pl.BlockSpec((HEADS, DH), lambda i: (0, 0)),
        ],
        out_specs=[
            pl.BlockSpec((FG, BN, FW), lambda i: (0, i, 0)),
            pl.BlockSpec((2 * HEADS, NP), lambda i: (0, 0)),
        ],
        out_shape=[
            jax.ShapeDtypeStruct((FG, NP, FW), F32),
            jax.ShapeDtypeStruct((2 * HEADS, NP), F32),
        ],
    )(x, scale1, shift1, W1, a1s, a1d)


def _mm2(h2, scale2, shift2, W2, a2):
    def body(h_ref, sc_ref, sh_ref, w_ref, a_ref, hp_ref, e_ref):
        i = pl.program_id(0)
        acc = jnp.zeros((BN, DOUT), F32)
        for f in range(FG):
            yb = h_ref[f] * sc_ref[f:f + 1, :] + sh_ref[f:f + 1, :]
            yb = jnp.maximum(yb, 0.01 * yb)
            acc = acc + lax.dot_general(
                yb, w_ref[f * FW:(f + 1) * FW, :], (((1,), (0,)), ((), ())),
                precision=HI)
        hp_ref[...] = acc
        e_ref[:, pl.ds(i * BN, BN)] = lax.dot_general(
            a_ref[...], acc, (((1,), (1,)), ((), ())), precision=HI)

    return pl.pallas_call(
        body,
        grid=(G,),
        in_specs=[
            pl.BlockSpec((FG, BN, FW), lambda i: (0, i, 0)),
            pl.BlockSpec((FG, FW), lambda i: (0, 0)),
            pl.BlockSpec((FG, FW), lambda i: (0, 0)),
            pl.BlockSpec((HEADS * DH, DOUT), lambda i: (0, 0)),
            pl.BlockSpec((2, DOUT), lambda i: (0, 0)),
        ],
        out_specs=[
            pl.BlockSpec((BN, DOUT), lambda i: (i, 0)),
            pl.BlockSpec((2, NP), lambda i: (0, 0)),
        ],
        out_shape=[
            jax.ShapeDtypeStruct((NP, DOUT), F32),
            jax.ShapeDtypeStruct((2, NP), F32),
        ],
    )(h2, scale2, shift2, W2, a2)


def _final(accA, accB, W_fc, b_fc):
    def body(a_ref, b_ref, w_ref, bias_ref, o_ref, h_ref):
        hf = a_ref[...] + b_ref[...]
        h_ref[...] = hf
        o_ref[...] = lax.dot_general(
            jnp.maximum(hf, 0.0), w_ref[...], (((1,), (0,)), ((), ())),
            precision=HI) + bias_ref[...]

    return pl.pallas_call(
        body,
        grid=(G,),
        in_specs=[
            pl.BlockSpec((BN, DOUT), lambda i: (i, 0)),
            pl.BlockSpec((BN, DOUT), lambda i: (i, 0)),
            pl.BlockSpec((DOUT, 2), lambda i: (0, 0)),
            pl.BlockSpec((1, 2), lambda i: (0, 0)),
        ],
        out_specs=[
            pl.BlockSpec((BN, 2), lambda i: (i, 0)),
            pl.BlockSpec((BN, DOUT), lambda i: (i, 0)),
        ],
        out_shape=[
            jax.ShapeDtypeStruct((NP, 2), F32),
            jax.ShapeDtypeStruct((NP, DOUT), F32),
        ],
    )(accA, accB, W_fc, b_fc)


def _leaky_exp(t):
    return jnp.exp(jnp.maximum(t, 0.2 * t))


def _init_ident(ident):
    def body(k, _):
        def vv(i, _):
            ident[k, pl.ds(i * NV, NV)] = (
                jnp.arange(NV, dtype=jnp.int32) + i * NV + k * 128)
            return 0
        lax.fori_loop(0, 8, vv, 0)
        return 0
    lax.fori_loop(0, 5, body, 0)


def _den_phase(s, src3, dst3, es_tab, ed_tab, den_buf, den_sh, ident):
    zv = jnp.zeros((NV,), F32)

    def zinit(i, _):
        den_buf[i, :] = zv
        return 0
    lax.fori_loop(0, NP // 16, zinit, 0)

    def chunk(j, _):
        for k in range(CH // NV):
            sv = src3[j, pl.ds(k * NV, NV)]
            dv = dst3[j, pl.ds(k * NV, NV)]
            es = plsc.load_gather(es_tab, [sv])
            ed = plsc.load_gather(ed_tab, [dv])
            z = _leaky_exp(es + ed)
            plsc.addupdate_scatter(
                den_buf, [lax.shift_right_logical(dv, 4),
                          lax.bitwise_and(dv, 15)], z)
        return 0
    lax.fori_loop(0, NCK, chunk, 0)

    @pl.when(s == 0)
    def _():
        pltpu.sync_copy(den_buf, den_sh)
    plsc.subcore_barrier()

    @pl.when(s != 0)
    def _():
        for k in range(5):
            pltpu.sync_copy(den_buf.at[pl.ds(k * 128, 128)],
                            den_sh.at[ident.at[k]], add=True)
    plsc.subcore_barrier()
    pltpu.sync_copy(den_sh, den_buf)


def _alpha_phase(src3, dst3, es_tab, ed_tab, den_buf, alpha_all):
    @plsc.parallel_loop(0, NCK, 1, unroll=2)
    def _(j):
        for k in range(CH // NV):
            sv = src3[j, pl.ds(k * NV, NV)]
            dv = dst3[j, pl.ds(k * NV, NV)]
            es = plsc.load_gather(es_tab, [sv])
            ed = plsc.load_gather(ed_tab, [dv])
            z = _leaky_exp(es + ed)
            den = plsc.load_gather(
                den_buf, [lax.shift_right_logical(dv, 4),
                          lax.bitwise_and(dv, 15)])
            alpha_all[j, pl.ds(k * NV, NV)] = z / (den + 1e-16)


def _zero_rows(rows):
    zv = jnp.zeros((NV,), F32)

    @plsc.parallel_loop(0, CH, 1, unroll=8)
    def _(i):
        for r in range(FW // NV):
            rows[i, pl.ds(r * NV, NV)] = zv


def _zero_acc_stripe(s, rows, acc_sh):
    base = s * ROWS_T
    for k in range(ROWS_T // CH):
        pltpu.sync_copy(rows, acc_sh.at[pl.ds(base + k * CH, CH)])


def _agg_chunk(j, hp_f, src3, dst3, alpha_all, rows, acc_sh, sem):
    pltpu.async_copy(hp_f.at[src3.at[j]], rows, sem).wait()
    _mul_scatter(j, rows, src3, dst3, alpha_all, acc_sh)


def _mul_scatter(j, rows, src3, dst3, alpha_all, acc_sh):
    _mul_rows(j, rows, alpha_all)
    pltpu.sync_copy(rows, acc_sh.at[dst3.at[j]], add=True)


def _mul_rows(j, rows, alpha_all):
    @plsc.parallel_loop(0, CH // NV, 1)
    def _(g):
        av = alpha_all[j, pl.ds(g * NV, NV)]
        for e in range(NV):
            a = av[e]
            i = g * NV + e
            for r in range(FW // NV):
                rows[i, pl.ds(r * NV, NV)] = rows[i, pl.ds(r * NV, NV)] * a


def _agg_pipe(hp_f, src3, dst3, alpha_all, rr, acc_sh, gs, ss):
    pltpu.async_copy(hp_f.at[src3.at[0]], rr[0], gs[0])

    def substep(j, b, tail):
        bn = (b + 1) % 3
        pltpu.make_async_copy(hp_f.at[src3.at[j]], rr[b], gs[b]).wait()

        @pl.when(j >= 2)
        def _():
            jm2 = jnp.maximum(j - 2, 0)
            pltpu.make_async_copy(rr[bn], acc_sh.at[dst3.at[jm2]],
                                  ss[bn]).wait()
        if not tail:
            pltpu.async_copy(hp_f.at[src3.at[j + 1]], rr[bn], gs[bn])
        _mul_rows(j, rr[b], alpha_all)
        pltpu.async_copy(rr[b], acc_sh.at[dst3.at[j]], ss[b], add=True)

    def step(t, _):
        j = 3 * t
        substep(j, 0, False)
        substep(j + 1, 1, False)
        substep(j + 2, 2, False)
        return 0
    lax.fori_loop(0, (NCK - 2) // 3, step, 0)
    substep(NCK - 2, (NCK - 2) % 3, False)
    substep(NCK - 1, (NCK - 1) % 3, True)
    for j in (NCK - 2, NCK - 1):
        b = j % 3
        pltpu.make_async_copy(rr[b], acc_sh.at[dst3.at[j]], ss[b]).wait()


def _sc_scratch():
    return [
        pltpu.VMEM((NCK, CH), jnp.int32),
        pltpu.VMEM((NCK, CH), jnp.int32),
        pltpu.VMEM((NP,), F32),
        pltpu.VMEM((NP,), F32),
        pltpu.VMEM((NP // 16, 16), F32),
        pltpu.VMEM((5, 128), jnp.int32),
        pltpu.VMEM((NCK, CH), F32),
        pltpu.VMEM((CH, FW), F32),
        pltpu.VMEM((CH, FW), F32),
        pltpu.VMEM((CH, FW), F32),
        pltpu.VMEM_SHARED((NP // 16, 16), F32),
        pltpu.VMEM_SHARED((NP, FW), F32),
        pltpu.SemaphoreType.DMA,
        pltpu.SemaphoreType.DMA,
        pltpu.SemaphoreType.DMA,
        pltpu.SemaphoreType.DMA,
        pltpu.SemaphoreType.DMA,
        pltpu.SemaphoreType.DMA,
    ]


_SC_PARAMS = pltpu.CompilerParams(use_tc_tiling_on_sc=False,
                                  needs_layout_passes=False)


def _make_sc1():
    mesh = plsc.VectorSubcoreMesh(core_axis_name="c", subcore_axis_name="s")

    @functools.partial(
        pl.kernel,
        out_type=jax.ShapeDtypeStruct((FG, NP, FW), F32),
        mesh=mesh,
        scratch_types=_sc_scratch(),
        compiler_params=_SC_PARAMS,
    )
    def sc1(src2d, dst2d, e1T, hp, out, src3, dst3, es_tab, ed_tab, den_buf,
            ident, alpha_all, rows0, rows1, rows2, den_sh, acc_sh, g0, g1, g2,
            s0, s1, s2):
        c = lax.axis_index("c")
        s = lax.axis_index("s")

        pltpu.sync_copy(src2d.at[pl.ds(s * NCK, NCK)], src3)
        pltpu.sync_copy(dst2d.at[pl.ds(s * NCK, NCK)], dst3)
        _init_ident(ident)

        def head(hh, _):
            h = c * 4 + hh
            pltpu.sync_copy(e1T.at[h], es_tab)
            pltpu.sync_copy(e1T.at[h + HEADS], ed_tab)
            _den_phase(s, src3, dst3, es_tab, ed_tab, den_buf, den_sh, ident)
            _alpha_phase(src3, dst3, es_tab, ed_tab, den_buf, alpha_all)
            for half in range(2):
                f = 2 * h + half
                _zero_rows(rows0)
                _zero_acc_stripe(s, rows0, acc_sh)
                plsc.subcore_barrier()
                _agg_pipe(hp.at[f], src3, dst3, alpha_all,
                          (rows0, rows1, rows2), acc_sh, (g0, g1, g2),
                          (s0, s1, s2))
                plsc.subcore_barrier()
                pltpu.sync_copy(
                    acc_sh.at[pl.ds(s * ROWS_T, ROWS_T)],
                    out.at[f].at[pl.ds(s * ROWS_T, ROWS_T)])
                plsc.subcore_barrier()
            return 0
        lax.fori_loop(0, 4, head, 0)

    return sc1


def _make_sc2():
    mesh = plsc.VectorSubcoreMesh(core_axis_name="c", subcore_axis_name="s")

    @functools.partial(
        pl.kernel,
        out_type=(jax.ShapeDtypeStruct((NP, DOUT), F32),
                  jax.ShapeDtypeStruct((NP, DOUT), F32)),
        mesh=mesh,
        scratch_types=_sc_scratch(),
        compiler_params=_SC_PARAMS,
    )
    def sc2(src2d, dst2d, e2T, hp2, outA, outB, src3, dst3, es_tab, ed_tab,
            den_buf, ident, alpha_all, rows0, rows1, rows2, den_sh, acc_sh,
            g0, g1, g2, s0, s1, s2):
        c = lax.axis_index("c")
        s = lax.axis_index("s")

        pltpu.sync_copy(src2d.at[pl.ds(s * NCK, NCK)], src3)
        pltpu.sync_copy(dst2d.at[pl.ds(s * NCK, NCK)], dst3)
        _init_ident(ident)

        pltpu.sync_copy(e2T.at[0], es_tab)
        pltpu.sync_copy(e2T.at[1], ed_tab)
        _den_phase(s, src3, dst3, es_tab, ed_tab, den_buf, den_sh, ident)
        _alpha_phase(src3, dst3, es_tab, ed_tab, den_buf, alpha_all)
        _zero_rows(rows0)
        _zero_acc_stripe(s, rows0, acc_sh)
        plsc.subcore_barrier()

        def chunk(j, _):
            @pl.when(lax.rem(j, 2) == c)
            def _():
                _agg_chunk(j, hp2, src3, dst3, alpha_all, rows0, acc_sh, g0)
            return 0
        lax.fori_loop(0, NCK, chunk, 0)
        plsc.subcore_barrier()

        @pl.when(c == 0)
        def _():
            pltpu.sync_copy(acc_sh.at[pl.ds(s * ROWS_T, ROWS_T)],
                            outA.at[pl.ds(s * ROWS_T, ROWS_T)])

        @pl.when(c == 1)
        def _():
            pltpu.sync_copy(acc_sh.at[pl.ds(s * ROWS_T, ROWS_T)],
                            outB.at[pl.ds(s * ROWS_T, ROWS_T)])

    return sc2


def kernel(x, adj, gamma1, beta1, W1, a1_src, a1_dst, gamma2, beta2, W2,
           a2_src, a2_dst, W_fc, b_fc):
    src2d = adj[0].reshape(E // CH, CH)
    dst2d = adj[1].reshape(E // CH, CH)
    xp = jnp.pad(x, ((0, NP - N), (0, 0)))

    st1 = _colstats(xp.reshape(1, NP, DIN))
    mean1 = st1[0, 0] / N
    var1 = st1[1, 0] / N - mean1 * mean1
    scale1 = gamma1 * lax.rsqrt(var1 + 1e-5)
    shift1 = beta1 - mean1 * scale1

    hp1, e1T = _mm1(xp, scale1.reshape(1, DIN), shift1.reshape(1, DIN), W1,
                    a1_src, a1_dst)
    h2 = _make_sc1()(src2d, dst2d, e1T, hp1)

    st2 = _colstats(h2)
    mean2 = st2[0] / N
    var2 = st2[1] / N - mean2 * mean2
    scale2 = gamma2.reshape(FG, FW) * lax.rsqrt(var2 + 1e-5)
    shift2 = beta2.reshape(FG, FW) - mean2 * scale2

    hp2, e2T = _mm2(h2, scale2, shift2, W2,
                    jnp.concatenate([a2_src, a2_dst], axis=0))
    accA, accB = _make_sc2()(src2d, dst2d, e2T, hp2)

    out, hfin = _final(accA, accB, W_fc, b_fc.reshape(1, 2))
    return (out[:N], hfin[:N])


# speedup vs baseline: 1.3912x; 1.3907x over previous
import functools

import jax
import jax.numpy as jnp
from jax import lax
from jax.experimental import pallas as pl
from jax.experimental.pallas import tpu as pltpu
from jax.experimental.pallas import tpu_sc as plsc

F32 = jnp.float32
HI = lax.Precision.HIGHEST

N = 10000
NP = 10240
E = 160000
DIN = 256
DH = 128
HEADS = 8
DOUT = 64

CH = 80
NCK = E // (16 * CH)
ECK = E // CH
ROWS_T = NP // 16
BN = 1280
G = NP // BN
NV = 16


def _colstats(a):
    H, n, D = a.shape

    def body(a_ref, o_ref):
        i = pl.program_id(0)

        @pl.when(i == 0)
        def _():
            o_ref[...] = jnp.zeros_like(o_ref)

        ab = a_ref[...]
        o_ref[0] += jnp.sum(ab, axis=1)
        o_ref[1] += jnp.sum(ab * ab, axis=1)

    return pl.pallas_call(
        body,
        grid=(n // BN,),
        in_specs=[pl.BlockSpec((H, BN, D), lambda i: (0, i, 0))],
        out_specs=pl.BlockSpec((2, H, D), lambda i: (0, 0, 0)),
        out_shape=jax.ShapeDtypeStruct((2, H, D), F32),
    )(a)


def _mm1(x, scale1, shift1, W1, a1s, a1d):
    def body(x_ref, sc_ref, sh_ref, w_ref, as_ref, ad_ref, hp_ref, e_ref):
        i = pl.program_id(0)
        hb = x_ref[...] * sc_ref[...] + sh_ref[...]
        for h in range(HEADS):
            wh = w_ref[:, h * DH:(h + 1) * DH]
            hph = lax.dot_general(hb, wh, (((1,), (0,)), ((), ())), precision=HI)
            hp_ref[h] = hph
            e_ref[h:h + 1, pl.ds(i * BN, BN)] = lax.dot_general(
                as_ref[h:h + 1, :], hph, (((1,), (1,)), ((), ())), precision=HI)
            e_ref[h + HEADS:h + HEADS + 1, pl.ds(i * BN, BN)] = lax.dot_general(
                ad_ref[h:h + 1, :], hph, (((1,), (1,)), ((), ())), precision=HI)

    return pl.pallas_call(
        body,
        grid=(G,),
        in_specs=[
            pl.BlockSpec((BN, DIN), lambda i: (i, 0)),
            pl.BlockSpec((1, DIN), lambda i: (0, 0)),
            pl.BlockSpec((1, DIN), lambda i: (0, 0)),
            pl.BlockSpec((DIN, HEADS * DH), lambda i: (0, 0)),
            pl.BlockSpec((HEADS, DH), lambda i: (0, 0)),
            pl.BlockSpec((HEADS, DH), lambda i: (0, 0)),
        ],
        out_specs=[
            pl.BlockSpec((HEADS, BN, DH), lambda i: (0, i, 0)),
            pl.BlockSpec((2 * HEADS, NP), lambda i: (0, 0)),
        ],
        out_shape=[
            jax.ShapeDtypeStruct((HEADS, NP, DH), F32),
            jax.ShapeDtypeStruct((2 * HEADS, NP), F32),
        ],
    )(x, scale1, shift1, W1, a1s, a1d)


def _mm2(h2, scale2, shift2, W2, a2):
    def body(h_ref, sc_ref, sh_ref, w_ref, a_ref, hp_ref, e_ref):
        i = pl.program_id(0)
        acc = jnp.zeros((BN, DOUT), F32)
        for h in range(HEADS):
            yb = h_ref[h] * sc_ref[h:h + 1, :] + sh_ref[h:h + 1, :]
            yb = jnp.maximum(yb, 0.01 * yb)
            acc = acc + lax.dot_general(
                yb, w_ref[h * DH:(h + 1) * DH, :], (((1,), (0,)), ((), ())),
                precision=HI)
        hp_ref[...] = acc
        e_ref[:, pl.ds(i * BN, BN)] = lax.dot_general(
            a_ref[...], acc, (((1,), (1,)), ((), ())), precision=HI)

    return pl.pallas_call(
        body,
        grid=(G,),
        in_specs=[
            pl.BlockSpec((HEADS, BN, DH), lambda i: (0, i, 0)),
            pl.BlockSpec((HEADS, DH), lambda i: (0, 0)),
            pl.BlockSpec((HEADS, DH), lambda i: (0, 0)),
            pl.BlockSpec((HEADS * DH, DOUT), lambda i: (0, 0)),
            pl.BlockSpec((2, DOUT), lambda i: (0, 0)),
        ],
        out_specs=[
            pl.BlockSpec((BN, DOUT), lambda i: (i, 0)),
            pl.BlockSpec((2, NP), lambda i: (0, 0)),
        ],
        out_shape=[
            jax.ShapeDtypeStruct((NP, DOUT), F32),
            jax.ShapeDtypeStruct((2, NP), F32),
        ],
    )(h2, scale2, shift2, W2, a2)


def _final(accA, accB, W_fc, b_fc):
    def body(a_ref, b_ref, w_ref, bias_ref, o_ref, h_ref):
        hf = a_ref[...] + b_ref[...]
        h_ref[...] = hf
        o_ref[...] = lax.dot_general(
            jnp.maximum(hf, 0.0), w_ref[...], (((1,), (0,)), ((), ())),
            precision=HI) + bias_ref[...]

    return pl.pallas_call(
        body,
        grid=(G,),
        in_specs=[
            pl.BlockSpec((BN, DOUT), lambda i: (i, 0)),
            pl.BlockSpec((BN, DOUT), lambda i: (i, 0)),
            pl.BlockSpec((DOUT, 2), lambda i: (0, 0)),
            pl.BlockSpec((1, 2), lambda i: (0, 0)),
        ],
        out_specs=[
            pl.BlockSpec((BN, 2), lambda i: (i, 0)),
            pl.BlockSpec((BN, DOUT), lambda i: (i, 0)),
        ],
        out_shape=[
            jax.ShapeDtypeStruct((NP, 2), F32),
            jax.ShapeDtypeStruct((NP, DOUT), F32),
        ],
    )(accA, accB, W_fc, b_fc)


def _leaky_exp(t):
    return jnp.exp(jnp.maximum(t, 0.2 * t))


def _init_ident(ident):
    def body(k, _):
        def vv(i, _):
            ident[k, pl.ds(i * NV, NV)] = (
                jnp.arange(NV, dtype=jnp.int32) + i * NV + k * 128)
            return 0
        lax.fori_loop(0, 8, vv, 0)
        return 0
    lax.fori_loop(0, 5, body, 0)


def _den_phase(s, src3, dst3, es_tab, ed_tab, den_buf, den_sh, ident):
    zv = jnp.zeros((NV,), F32)

    def zinit(i, _):
        den_buf[i, :] = zv
        return 0
    lax.fori_loop(0, NP // 16, zinit, 0)

    def chunk(j, _):
        for k in range(CH // NV):
            sv = src3[j, pl.ds(k * NV, NV)]
            dv = dst3[j, pl.ds(k * NV, NV)]
            es = plsc.load_gather(es_tab, [sv])
            ed = plsc.load_gather(ed_tab, [dv])
            z = _leaky_exp(es + ed)
            plsc.addupdate_scatter(
                den_buf, [lax.shift_right_logical(dv, 4),
                          lax.bitwise_and(dv, 15)], z)
        return 0
    lax.fori_loop(0, NCK, chunk, 0)

    @pl.when(s == 0)
    def _():
        pltpu.sync_copy(den_buf, den_sh)
    plsc.subcore_barrier()

    @pl.when(s != 0)
    def _():
        for k in range(5):
            pltpu.sync_copy(den_buf.at[pl.ds(k * 128, 128)],
                            den_sh.at[ident.at[k]], add=True)
    plsc.subcore_barrier()
    pltpu.sync_copy(den_sh, den_buf)


def _alpha_phase(src3, dst3, es_tab, ed_tab, den_buf, alpha_all):
    @plsc.parallel_loop(0, NCK, 1, unroll=2)
    def _(j):
        for k in range(CH // NV):
            sv = src3[j, pl.ds(k * NV, NV)]
            dv = dst3[j, pl.ds(k * NV, NV)]
            es = plsc.load_gather(es_tab, [sv])
            ed = plsc.load_gather(ed_tab, [dv])
            z = _leaky_exp(es + ed)
            den = plsc.load_gather(
                den_buf, [lax.shift_right_logical(dv, 4),
                          lax.bitwise_and(dv, 15)])
            alpha_all[j, pl.ds(k * NV, NV)] = z / (den + 1e-16)


def _mul_rows(rows, al, width):
    @plsc.parallel_loop(0, CH // NV, 1)
    def _(g):
        av = al[pl.ds(g * NV, NV)]
        for e in range(NV):
            a = av[e]
            i = g * NV + e
            for r in range(width // NV):
                rows[i, pl.ds(r * NV, NV)] = rows[i, pl.ds(r * NV, NV)] * a


def _zero_rows(rows, width):
    zv = jnp.zeros((NV,), F32)

    @plsc.parallel_loop(0, CH, 1, unroll=8)
    def _(i):
        for r in range(width // NV):
            rows[i, pl.ds(r * NV, NV)] = zv


def _zero_acc_stripe(s, rows, acc_sh):
    base = s * ROWS_T
    for k in range(ROWS_T // CH):
        pltpu.sync_copy(rows, acc_sh.at[pl.ds(base + k * CH, CH)])


_SC_PARAMS = pltpu.CompilerParams(use_tc_tiling_on_sc=False,
                                  needs_layout_passes=False)


def _make_sc1a():
    mesh = plsc.VectorSubcoreMesh(core_axis_name="c", subcore_axis_name="s")

    @functools.partial(
        pl.kernel,
        out_type=jax.ShapeDtypeStruct((HEADS, ECK, CH), F32),
        mesh=mesh,
        scratch_types=[
            pltpu.VMEM((NCK, CH), jnp.int32),
            pltpu.VMEM((NCK, CH), jnp.int32),
            pltpu.VMEM((NP,), F32),
            pltpu.VMEM((NP,), F32),
            pltpu.VMEM((NP // 16, 16), F32),
            pltpu.VMEM((5, 128), jnp.int32),
            pltpu.VMEM((NCK, CH), F32),
            pltpu.VMEM_SHARED((NP // 16, 16), F32),
        ],
        compiler_params=_SC_PARAMS,
    )
    def sc1a(src2d, dst2d, e1T, alpha_out, src3, dst3, es_tab, ed_tab,
             den_buf, ident, alpha_all, den_sh):
        c = lax.axis_index("c")
        s = lax.axis_index("s")

        pltpu.sync_copy(src2d.at[pl.ds(s * NCK, NCK)], src3)
        pltpu.sync_copy(dst2d.at[pl.ds(s * NCK, NCK)], dst3)
        _init_ident(ident)

        def head(hh, _):
            h = c * 4 + hh
            pltpu.sync_copy(e1T.at[h], es_tab)
            pltpu.sync_copy(e1T.at[h + HEADS], ed_tab)
            _den_phase(s, src3, dst3, es_tab, ed_tab, den_buf, den_sh, ident)
            _alpha_phase(src3, dst3, es_tab, ed_tab, den_buf, alpha_all)
            pltpu.sync_copy(alpha_all,
                            alpha_out.at[h].at[pl.ds(s * NCK, NCK)])
            return 0
        lax.fori_loop(0, 4, head, 0)

    return sc1a


def _make_sc1b():
    mesh = plsc.VectorSubcoreMesh(core_axis_name="c", subcore_axis_name="s")

    @functools.partial(
        pl.kernel,
        out_type=jax.ShapeDtypeStruct((HEADS, NP, DH), F32),
        mesh=mesh,
        scratch_types=[
            pltpu.VMEM((NCK, CH), jnp.int32),
            pltpu.VMEM((NCK, CH), jnp.int32),
            pltpu.VMEM((CH, DH), F32),
            pltpu.VMEM((CH, DH), F32),
            pltpu.VMEM((CH,), F32),
            pltpu.VMEM((CH,), F32),
            pltpu.VMEM_SHARED((NP, DH), F32),
            pltpu.SemaphoreType.DMA,
            pltpu.SemaphoreType.DMA,
            pltpu.SemaphoreType.DMA,
            pltpu.SemaphoreType.DMA,
        ],
        compiler_params=_SC_PARAMS,
    )
    def sc1b(src2d, dst2d, alpha_hbm, hp, out, src3, dst3, rows0, rows1,
             al0, al1, acc_sh, g0, g1, a0, a1):
        c = lax.axis_index("c")
        s = lax.axis_index("s")
        rr = (rows0, rows1)
        aa = (al0, al1)
        gs = (g0, g1)
        asems = (a0, a1)

        pltpu.sync_copy(src2d.at[pl.ds(s * NCK, NCK)], src3)
        pltpu.sync_copy(dst2d.at[pl.ds(s * NCK, NCK)], dst3)

        def head(hh, _):
            h = c * 4 + hh
            alpha_h = alpha_hbm.at[h]
            hp_h = hp.at[h]
            _zero_rows(rows0, DH)
            _zero_acc_stripe(s, rows0, acc_sh)
            plsc.subcore_barrier()

            pltpu.async_copy(hp_h.at[src3.at[0]], rows0, g0)
            pltpu.async_copy(alpha_h.at[s * NCK], al0, a0)

            def substep(j, b, last):
                bn = 1 - b
                pltpu.make_async_copy(hp_h.at[src3.at[j]], rr[b],
                                      gs[b]).wait()
                pltpu.make_async_copy(alpha_h.at[s * NCK + j], aa[b],
                                      asems[b]).wait()
                if not last:
                    pltpu.async_copy(hp_h.at[src3.at[j + 1]], rr[bn], gs[bn])
                    pltpu.async_copy(alpha_h.at[s * NCK + j + 1], aa[bn],
                                     asems[bn])
                _mul_rows(rr[b], aa[b], DH)
                pltpu.sync_copy(rr[b], acc_sh.at[dst3.at[j]], add=True)

            def step(t, _):
                substep(2 * t, 0, False)
                substep(2 * t + 1, 1, False)
                return 0
            lax.fori_loop(0, (NCK - 1) // 2, step, 0)
            substep(NCK - 1, (NCK - 1) % 2, True)

            plsc.subcore_barrier()
            pltpu.sync_copy(acc_sh.at[pl.ds(s * ROWS_T, ROWS_T)],
                            out.at[h].at[pl.ds(s * ROWS_T, ROWS_T)])
            plsc.subcore_barrier()
            return 0
        lax.fori_loop(0, 4, head, 0)

    return sc1b


def _make_sc2():
    mesh = plsc.VectorSubcoreMesh(core_axis_name="c", subcore_axis_name="s")

    @functools.partial(
        pl.kernel,
        out_type=(jax.ShapeDtypeStruct((NP, DOUT), F32),
                  jax.ShapeDtypeStruct((NP, DOUT), F32)),
        mesh=mesh,
        scratch_types=[
            pltpu.VMEM((NCK, CH), jnp.int32),
            pltpu.VMEM((NCK, CH), jnp.int32),
            pltpu.VMEM((NP,), F32),
            pltpu.VMEM((NP,), F32),
            pltpu.VMEM((NP // 16, 16), F32),
            pltpu.VMEM((5, 128), jnp.int32),
            pltpu.VMEM((NCK, CH), F32),
            pltpu.VMEM((CH, DOUT), F32),
            pltpu.VMEM_SHARED((NP // 16, 16), F32),
            pltpu.VMEM_SHARED((NP, DOUT), F32),
            pltpu.SemaphoreType.DMA,
        ],
        compiler_params=_SC_PARAMS,
    )
    def sc2(src2d, dst2d, e2T, hp2, outA, outB, src3, dst3, es_tab, ed_tab,
            den_buf, ident, alpha_all, rows0, den_sh, acc_sh, g0):
        c = lax.axis_index("c")
        s = lax.axis_index("s")

        pltpu.sync_copy(src2d.at[pl.ds(s * NCK, NCK)], src3)
        pltpu.sync_copy(dst2d.at[pl.ds(s * NCK, NCK)], dst3)
        _init_ident(ident)

        pltpu.sync_copy(e2T.at[0], es_tab)
        pltpu.sync_copy(e2T.at[1], ed_tab)
        _den_phase(s, src3, dst3, es_tab, ed_tab, den_buf, den_sh, ident)
        _alpha_phase(src3, dst3, es_tab, ed_tab, den_buf, alpha_all)
        _zero_rows(rows0, DOUT)
        _zero_acc_stripe(s, rows0, acc_sh)
        plsc.subcore_barrier()

        def chunk(j, _):
            @pl.when(lax.rem(j, 2) == c)
            def _():
                pltpu.async_copy(hp2.at[src3.at[j]], rows0, g0).wait()
                _mul_rows(rows0, alpha_all.at[j], DOUT)
                pltpu.sync_copy(rows0, acc_sh.at[dst3.at[j]], add=True)
            return 0
        lax.fori_loop(0, NCK, chunk, 0)
        plsc.subcore_barrier()

        @pl.when(c == 0)
        def _():
            pltpu.sync_copy(acc_sh.at[pl.ds(s * ROWS_T, ROWS_T)],
                            outA.at[pl.ds(s * ROWS_T, ROWS_T)])

        @pl.when(c == 1)
        def _():
            pltpu.sync_copy(acc_sh.at[pl.ds(s * ROWS_T, ROWS_T)],
                            outB.at[pl.ds(s * ROWS_T, ROWS_T)])

    return sc2


def kernel(x, adj, gamma1, beta1, W1, a1_src, a1_dst, gamma2, beta2, W2,
           a2_src, a2_dst, W_fc, b_fc):
    src2d = adj[0].reshape(ECK, CH)
    dst2d = adj[1].reshape(ECK, CH)
    xp = jnp.pad(x, ((0, NP - N), (0, 0)))

    st1 = _colstats(xp.reshape(1, NP, DIN))
    mean1 = st1[0, 0] / N
    var1 = st1[1, 0] / N - mean1 * mean1
    scale1 = gamma1 * lax.rsqrt(var1 + 1e-5)
    shift1 = beta1 - mean1 * scale1

    hp1, e1T = _mm1(xp, scale1.reshape(1, DIN), shift1.reshape(1, DIN), W1,
                    a1_src, a1_dst)
    alpha1 = _make_sc1a()(src2d, dst2d, e1T)
    h2 = _make_sc1b()(src2d, dst2d, alpha1, hp1)

    st2 = _colstats(h2)
    mean2 = st2[0] / N
    var2 = st2[1] / N - mean2 * mean2
    scale2 = gamma2.reshape(HEADS, DH) * lax.rsqrt(var2 + 1e-5)
    shift2 = beta2.reshape(HEADS, DH) - mean2 * scale2

    hp2, e2T = _mm2(h2, scale2, shift2, W2,
                    jnp.concatenate([a2_src, a2_dst], axis=0))
    accA, accB = _make_sc2()(src2d, dst2d, e2T, hp2)

    out, hfin = _final(accA, accB, W_fc, b_fc.reshape(1, 2))
    return (out[:N], hfin[:N])


# speedup vs baseline: 1.4243x; 1.0238x over previous
import functools

import jax
import jax.numpy as jnp
from jax import lax
from jax.experimental import pallas as pl
from jax.experimental.pallas import tpu as pltpu
from jax.experimental.pallas import tpu_sc as plsc

F32 = jnp.float32
HI = lax.Precision.HIGHEST

N = 10000
NP = 10240
E = 160000
DIN = 256
DH = 128
HEADS = 8
DOUT = 64

CH = 80
NCK = E // (16 * CH)
ECK = E // CH
ROWS_T = NP // 16
BN = 1280
G = NP // BN
NV = 16


def _colstats(a):
    H, n, D = a.shape

    def body(a_ref, o_ref):
        i = pl.program_id(0)

        @pl.when(i == 0)
        def _():
            o_ref[...] = jnp.zeros_like(o_ref)

        ab = a_ref[...]
        o_ref[0] += jnp.sum(ab, axis=1)
        o_ref[1] += jnp.sum(ab * ab, axis=1)

    return pl.pallas_call(
        body,
        grid=(n // BN,),
        in_specs=[pl.BlockSpec((H, BN, D), lambda i: (0, i, 0))],
        out_specs=pl.BlockSpec((2, H, D), lambda i: (0, 0, 0)),
        out_shape=jax.ShapeDtypeStruct((2, H, D), F32),
    )(a)


def _mm1(x, scale1, shift1, W1, a1s, a1d):
    def body(x_ref, sc_ref, sh_ref, w_ref, as_ref, ad_ref, hp_ref, e_ref):
        i = pl.program_id(0)
        hb = x_ref[...] * sc_ref[...] + sh_ref[...]
        for h in range(HEADS):
            wh = w_ref[:, h * DH:(h + 1) * DH]
            hph = lax.dot_general(hb, wh, (((1,), (0,)), ((), ())), precision=HI)
            hp_ref[h] = hph
            e_ref[h:h + 1, pl.ds(i * BN, BN)] = lax.dot_general(
                as_ref[h:h + 1, :], hph, (((1,), (1,)), ((), ())), precision=HI)
            e_ref[h + HEADS:h + HEADS + 1, pl.ds(i * BN, BN)] = lax.dot_general(
                ad_ref[h:h + 1, :], hph, (((1,), (1,)), ((), ())), precision=HI)

    return pl.pallas_call(
        body,
        grid=(G,),
        in_specs=[
            pl.BlockSpec((BN, DIN), lambda i: (i, 0)),
            pl.BlockSpec((1, DIN), lambda i: (0, 0)),
            pl.BlockSpec((1, DIN), lambda i: (0, 0)),
            pl.BlockSpec((DIN, HEADS * DH), lambda i: (0, 0)),
            pl.BlockSpec((HEADS, DH), lambda i: (0, 0)),
            pl.BlockSpec((HEADS, DH), lambda i: (0, 0)),
        ],
        out_specs=[
            pl.BlockSpec((HEADS, BN, DH), lambda i: (0, i, 0)),
            pl.BlockSpec((2 * HEADS, NP), lambda i: (0, 0)),
        ],
        out_shape=[
            jax.ShapeDtypeStruct((HEADS, NP, DH), F32),
            jax.ShapeDtypeStruct((2 * HEADS, NP), F32),
        ],
    )(x, scale1, shift1, W1, a1s, a1d)


def _mm2(h2, scale2, shift2, W2, a2):
    def body(h_ref, sc_ref, sh_ref, w_ref, a_ref, hp_ref, e_ref):
        i = pl.program_id(0)
        acc = jnp.zeros((BN, DOUT), F32)
        for h in range(HEADS):
            yb = h_ref[h] * sc_ref[h:h + 1, :] + sh_ref[h:h + 1, :]
            yb = jnp.maximum(yb, 0.01 * yb)
            acc = acc + lax.dot_general(
                yb, w_ref[h * DH:(h + 1) * DH, :], (((1,), (0,)), ((), ())),
                precision=HI)
        hp_ref[...] = acc
        e_ref[:, pl.ds(i * BN, BN)] = lax.dot_general(
            a_ref[...], acc, (((1,), (1,)), ((), ())), precision=HI)

    return pl.pallas_call(
        body,
        grid=(G,),
        in_specs=[
            pl.BlockSpec((HEADS, BN, DH), lambda i: (0, i, 0)),
            pl.BlockSpec((HEADS, DH), lambda i: (0, 0)),
            pl.BlockSpec((HEADS, DH), lambda i: (0, 0)),
            pl.BlockSpec((HEADS * DH, DOUT), lambda i: (0, 0)),
            pl.BlockSpec((2, DOUT), lambda i: (0, 0)),
        ],
        out_specs=[
            pl.BlockSpec((BN, DOUT), lambda i: (i, 0)),
            pl.BlockSpec((2, NP), lambda i: (0, 0)),
        ],
        out_shape=[
            jax.ShapeDtypeStruct((NP, DOUT), F32),
            jax.ShapeDtypeStruct((2, NP), F32),
        ],
    )(h2, scale2, shift2, W2, a2)


def _final(accA, accB, W_fc, b_fc):
    def body(a_ref, b_ref, w_ref, bias_ref, o_ref, h_ref):
        hf = a_ref[...] + b_ref[...]
        h_ref[...] = hf
        o_ref[...] = lax.dot_general(
            jnp.maximum(hf, 0.0), w_ref[...], (((1,), (0,)), ((), ())),
            precision=HI) + bias_ref[...]

    return pl.pallas_call(
        body,
        grid=(G,),
        in_specs=[
            pl.BlockSpec((BN, DOUT), lambda i: (i, 0)),
            pl.BlockSpec((BN, DOUT), lambda i: (i, 0)),
            pl.BlockSpec((DOUT, 2), lambda i: (0, 0)),
            pl.BlockSpec((1, 2), lambda i: (0, 0)),
        ],
        out_specs=[
            pl.BlockSpec((BN, 2), lambda i: (i, 0)),
            pl.BlockSpec((BN, DOUT), lambda i: (i, 0)),
        ],
        out_shape=[
            jax.ShapeDtypeStruct((NP, 2), F32),
            jax.ShapeDtypeStruct((NP, DOUT), F32),
        ],
    )(accA, accB, W_fc, b_fc)


def _leaky_exp(t):
    return jnp.exp(jnp.maximum(t, 0.2 * t))


def _init_ident(ident):
    def body(k, _):
        def vv(i, _):
            ident[k, pl.ds(i * NV, NV)] = (
                jnp.arange(NV, dtype=jnp.int32) + i * NV + k * 128)
            return 0
        lax.fori_loop(0, 8, vv, 0)
        return 0
    lax.fori_loop(0, 5, body, 0)


def _den_phase(s, src3, dst3, es_tab, ed_tab, den_buf, den_sh, ident):
    zv = jnp.zeros((NV,), F32)

    def zinit(i, _):
        den_buf[i, :] = zv
        return 0
    lax.fori_loop(0, NP // 16, zinit, 0)

    def chunk(j, _):
        for k in range(CH // NV):
            sv = src3[j, pl.ds(k * NV, NV)]
            dv = dst3[j, pl.ds(k * NV, NV)]
            es = plsc.load_gather(es_tab, [sv])
            ed = plsc.load_gather(ed_tab, [dv])
            z = _leaky_exp(es + ed)
            plsc.addupdate_scatter(
                den_buf, [lax.shift_right_logical(dv, 4),
                          lax.bitwise_and(dv, 15)], z)
        return 0
    lax.fori_loop(0, NCK, chunk, 0)

    @pl.when(s == 0)
    def _():
        pltpu.sync_copy(den_buf, den_sh)
    plsc.subcore_barrier()

    @pl.when(s != 0)
    def _():
        for k in range(5):
            pltpu.sync_copy(den_buf.at[pl.ds(k * 128, 128)],
                            den_sh.at[ident.at[k]], add=True)
    plsc.subcore_barrier()
    pltpu.sync_copy(den_sh, den_buf)


def _alpha_phase(src3, dst3, es_tab, ed_tab, den_buf, alpha_all):
    @plsc.parallel_loop(0, NCK, 1, unroll=2)
    def _(j):
        for k in range(CH // NV):
            sv = src3[j, pl.ds(k * NV, NV)]
            dv = dst3[j, pl.ds(k * NV, NV)]
            es = plsc.load_gather(es_tab, [sv])
            ed = plsc.load_gather(ed_tab, [dv])
            z = _leaky_exp(es + ed)
            den = plsc.load_gather(
                den_buf, [lax.shift_right_logical(dv, 4),
                          lax.bitwise_and(dv, 15)])
            alpha_all[j, pl.ds(k * NV, NV)] = z / (den + 1e-16)


def _mul_rows(rows, al, width):
    @plsc.parallel_loop(0, CH // NV, 1)
    def _(g):
        av = al[pl.ds(g * NV, NV)]
        for e in range(NV):
            a = av[e]
            i = g * NV + e
            for r in range(width // NV):
                rows[i, pl.ds(r * NV, NV)] = rows[i, pl.ds(r * NV, NV)] * a


def _zero_rows(rows, width):
    zv = jnp.zeros((NV,), F32)

    @plsc.parallel_loop(0, CH, 1, unroll=8)
    def _(i):
        for r in range(width // NV):
            rows[i, pl.ds(r * NV, NV)] = zv


def _zero_acc_stripe(s, rows, acc_sh):
    base = s * ROWS_T
    for k in range(ROWS_T // CH):
        pltpu.sync_copy(rows, acc_sh.at[pl.ds(base + k * CH, CH)])


_SC_PARAMS = pltpu.CompilerParams(use_tc_tiling_on_sc=False,
                                  needs_layout_passes=False)


def _make_sc1a():
    mesh = plsc.VectorSubcoreMesh(core_axis_name="c", subcore_axis_name="s")

    @functools.partial(
        pl.kernel,
        out_type=jax.ShapeDtypeStruct((HEADS, ECK, CH), F32),
        mesh=mesh,
        scratch_types=[
            pltpu.VMEM((NCK, CH), jnp.int32),
            pltpu.VMEM((NCK, CH), jnp.int32),
            pltpu.VMEM((NP,), F32),
            pltpu.VMEM((NP,), F32),
            pltpu.VMEM((NP // 16, 16), F32),
            pltpu.VMEM((5, 128), jnp.int32),
            pltpu.VMEM((NCK, CH), F32),
            pltpu.VMEM_SHARED((NP // 16, 16), F32),
        ],
        compiler_params=_SC_PARAMS,
    )
    def sc1a(src2d, dst2d, e1T, alpha_out, src3, dst3, es_tab, ed_tab,
             den_buf, ident, alpha_all, den_sh):
        c = lax.axis_index("c")
        s = lax.axis_index("s")

        pltpu.sync_copy(src2d.at[pl.ds(s * NCK, NCK)], src3)
        pltpu.sync_copy(dst2d.at[pl.ds(s * NCK, NCK)], dst3)
        _init_ident(ident)

        def head(hh, _):
            h = c * 4 + hh
            pltpu.sync_copy(e1T.at[h], es_tab)
            pltpu.sync_copy(e1T.at[h + HEADS], ed_tab)
            _den_phase(s, src3, dst3, es_tab, ed_tab, den_buf, den_sh, ident)
            _alpha_phase(src3, dst3, es_tab, ed_tab, den_buf, alpha_all)
            pltpu.sync_copy(alpha_all,
                            alpha_out.at[h].at[pl.ds(s * NCK, NCK)])
            return 0
        lax.fori_loop(0, 4, head, 0)

    return sc1a


def _make_sc1b():
    mesh = plsc.VectorSubcoreMesh(core_axis_name="c", subcore_axis_name="s")

    @functools.partial(
        pl.kernel,
        out_type=jax.ShapeDtypeStruct((HEADS, NP, DH), F32),
        mesh=mesh,
        scratch_types=[
            pltpu.VMEM((NCK, CH), jnp.int32),
            pltpu.VMEM((NCK, CH), jnp.int32),
            pltpu.VMEM((CH, DH), F32),
            pltpu.VMEM((CH, DH), F32),
            pltpu.VMEM((CH,), F32),
            pltpu.VMEM((CH,), F32),
            pltpu.VMEM_SHARED((NP, DH), F32),
            pltpu.SemaphoreType.DMA,
            pltpu.SemaphoreType.DMA,
            pltpu.SemaphoreType.DMA,
            pltpu.SemaphoreType.DMA,
        ],
        compiler_params=_SC_PARAMS,
    )
    def sc1b(src2d, dst2d, alpha_hbm, hp, out, src3, dst3, rows0, rows1,
             al0, al1, acc_sh, g0, g1, a0, a1):
        c = lax.axis_index("c")
        s = lax.axis_index("s")
        rr = (rows0, rows1)
        aa = (al0, al1)
        gs = (g0, g1)
        asems = (a0, a1)

        pltpu.sync_copy(src2d.at[pl.ds(s * NCK, NCK)], src3)
        pltpu.sync_copy(dst2d.at[pl.ds(s * NCK, NCK)], dst3)

        def head(hh, _):
            h = c * 4 + hh
            alpha_h = alpha_hbm.at[h]
            hp_h = hp.at[h]
            _zero_rows(rows0, DH)
            _zero_acc_stripe(s, rows0, acc_sh)
            plsc.subcore_barrier()

            pltpu.async_copy(hp_h.at[src3.at[0]], rows0, g0)
            pltpu.async_copy(alpha_h.at[s * NCK], al0, a0)

            def substep(j, b, last):
                bn = 1 - b
                pltpu.make_async_copy(hp_h.at[src3.at[j]], rr[b],
                                      gs[b]).wait()
                pltpu.make_async_copy(alpha_h.at[s * NCK + j], aa[b],
                                      asems[b]).wait()
                if not last:
                    pltpu.async_copy(hp_h.at[src3.at[j + 1]], rr[bn], gs[bn])
                    pltpu.async_copy(alpha_h.at[s * NCK + j + 1], aa[bn],
                                     asems[bn])
                _mul_rows(rr[b], aa[b], DH)
                pltpu.sync_copy(rr[b], acc_sh.at[dst3.at[j]], add=True)

            def step(t, _):
                substep(2 * t, 0, False)
                substep(2 * t + 1, 1, False)
                return 0
            lax.fori_loop(0, (NCK - 1) // 2, step, 0)
            substep(NCK - 1, (NCK - 1) % 2, True)

            plsc.subcore_barrier()
            pltpu.sync_copy(acc_sh.at[pl.ds(s * ROWS_T, ROWS_T)],
                            out.at[h].at[pl.ds(s * ROWS_T, ROWS_T)])
            plsc.subcore_barrier()
            return 0
        lax.fori_loop(0, 4, head, 0)

    return sc1b


def _make_sc2():
    mesh = plsc.VectorSubcoreMesh(core_axis_name="c", subcore_axis_name="s")

    @functools.partial(
        pl.kernel,
        out_type=(jax.ShapeDtypeStruct((NP, DOUT), F32),
                  jax.ShapeDtypeStruct((NP, DOUT), F32)),
        mesh=mesh,
        scratch_types=[
            pltpu.VMEM((NCK, CH), jnp.int32),
            pltpu.VMEM((NCK, CH), jnp.int32),
            pltpu.VMEM((NP,), F32),
            pltpu.VMEM((NP,), F32),
            pltpu.VMEM((NP // 16, 16), F32),
            pltpu.VMEM((5, 128), jnp.int32),
            pltpu.VMEM((NCK, CH), F32),
            pltpu.VMEM((CH, DOUT), F32),
            pltpu.VMEM((CH, DOUT), F32),
            pltpu.VMEM_SHARED((NP // 16, 16), F32),
            pltpu.VMEM_SHARED((NP, DOUT), F32),
            pltpu.SemaphoreType.DMA,
            pltpu.SemaphoreType.DMA,
        ],
        compiler_params=_SC_PARAMS,
    )
    def sc2(src2d, dst2d, e2T, hp2, outA, outB, src3, dst3, es_tab, ed_tab,
            den_buf, ident, alpha_all, rows0, rows1, den_sh, acc_sh, g0, g1):
        c = lax.axis_index("c")
        s = lax.axis_index("s")

        pltpu.sync_copy(src2d.at[pl.ds(s * NCK, NCK)], src3)
        pltpu.sync_copy(dst2d.at[pl.ds(s * NCK, NCK)], dst3)
        _init_ident(ident)

        pltpu.sync_copy(e2T.at[0], es_tab)
        pltpu.sync_copy(e2T.at[1], ed_tab)
        _den_phase(s, src3, dst3, es_tab, ed_tab, den_buf, den_sh, ident)
        _alpha_phase(src3, dst3, es_tab, ed_tab, den_buf, alpha_all)
        _zero_rows(rows0, DOUT)
        _zero_acc_stripe(s, rows0, acc_sh)
        plsc.subcore_barrier()

        rr = (rows0, rows1)
        gs = (g0, g1)
        pltpu.async_copy(hp2.at[src3.at[c]], rows0, g0)

        def substep(t, b):
            j = 2 * t + c
            pltpu.make_async_copy(hp2.at[src3.at[j]], rr[b], gs[b]).wait()

            @pl.when(j + 2 < NCK)
            def _():
                jn = jnp.minimum(j + 2, NCK - 1)
                pltpu.async_copy(hp2.at[src3.at[jn]], rr[1 - b], gs[1 - b])
            _mul_rows(rr[b], alpha_all.at[j], DOUT)
            pltpu.sync_copy(rr[b], acc_sh.at[dst3.at[j]], add=True)

        def step(tt, _):
            substep(2 * tt, 0)
            substep(2 * tt + 1, 1)
            return 0
        lax.fori_loop(0, 31, step, 0)

        @pl.when(c == 0)
        def _():
            substep(62, 0)
        plsc.subcore_barrier()

        @pl.when(c == 0)
        def _():
            pltpu.sync_copy(acc_sh.at[pl.ds(s * ROWS_T, ROWS_T)],
                            outA.at[pl.ds(s * ROWS_T, ROWS_T)])

        @pl.when(c == 1)
        def _():
            pltpu.sync_copy(acc_sh.at[pl.ds(s * ROWS_T, ROWS_T)],
                            outB.at[pl.ds(s * ROWS_T, ROWS_T)])

    return sc2


def kernel(x, adj, gamma1, beta1, W1, a1_src, a1_dst, gamma2, beta2, W2,
           a2_src, a2_dst, W_fc, b_fc):
    src2d = adj[0].reshape(ECK, CH)
    dst2d = adj[1].reshape(ECK, CH)
    xp = jnp.pad(x, ((0, NP - N), (0, 0)))

    st1 = _colstats(xp.reshape(1, NP, DIN))
    mean1 = st1[0, 0] / N
    var1 = st1[1, 0] / N - mean1 * mean1
    scale1 = gamma1 * lax.rsqrt(var1 + 1e-5)
    shift1 = beta1 - mean1 * scale1

    hp1, e1T = _mm1(xp, scale1.reshape(1, DIN), shift1.reshape(1, DIN), W1,
                    a1_src, a1_dst)
    alpha1 = _make_sc1a()(src2d, dst2d, e1T)
    h2 = _make_sc1b()(src2d, dst2d, alpha1, hp1)

    st2 = _colstats(h2)
    mean2 = st2[0] / N
    var2 = st2[1] / N - mean2 * mean2
    scale2 = gamma2.reshape(HEADS, DH) * lax.rsqrt(var2 + 1e-5)
    shift2 = beta2.reshape(HEADS, DH) - mean2 * scale2

    hp2, e2T = _mm2(h2, scale2, shift2, W2,
                    jnp.concatenate([a2_src, a2_dst], axis=0))
    accA, accB = _make_sc2()(src2d, dst2d, e2T, hp2)

    out, hfin = _final(accA, accB, W_fc, b_fc.reshape(1, 2))
    return (out[:N], hfin[:N])
